# baseline (speedup 1.0000x reference)
import functools

import jax
import jax.numpy as jnp
from jax import lax
from jax.experimental import pallas as pl
from jax.experimental.pallas import tpu as pltpu
from jax.experimental.pallas import tpu_sc as plsc

N_CAT = 8
VOCAB = 1000
EMBED = 64
OUT_DIM = 512
LANE = 128

NC, NS = 2, 16
NW = NC * NS
CROWS = 128


def _gather_body(tab_hbm, idx_hbm, cat_hbm, idx_v, rows_v, sem_i, sem_g, sem_s):
    batch = cat_hbm.shape[1]
    rows_per_w = batch // NW
    kchunks = rows_per_w // CROWS
    nunits = N_CAT * kchunks
    wid = lax.axis_index("s") * NC + lax.axis_index("c")
    r0 = wid * rows_per_w

    pltpu.async_copy(idx_hbm.at[:, pl.ds(r0, rows_per_w)], idx_v, sem_i).wait()

    def gather(u, buf):
        i, k = u // kchunks, u % kchunks
        return pltpu.async_copy(
            tab_hbm.at[idx_v.at[i, pl.ds(k * CROWS, CROWS)]],
            rows_v.at[buf], sem_g)

    def store(u, buf):
        i, k = u // kchunks, u % kchunks
        return pltpu.async_copy(
            rows_v.at[buf], cat_hbm.at[i, pl.ds(r0 + k * CROWS, CROWS)], sem_s)

    g = gather(0, 0)
    st = None
    for u in range(nunits):
        cur, nxt = u % 2, (u + 1) % 2
        g.wait()
        if u + 1 < nunits:
            if st is not None:
                st.wait()
            g = gather(u + 1, nxt)
        st = store(u, cur)
    st.wait()


def _sc_gather(tab, idxt):
    batch = idxt.shape[1]
    return pl.kernel(
        _gather_body,
        out_type=jax.ShapeDtypeStruct((N_CAT, batch, LANE), jnp.float32),
        mesh=plsc.VectorSubcoreMesh(
            core_axis_name="c", subcore_axis_name="s",
            num_cores=NC, num_subcores=NS),
        scratch_types=[
            pltpu.VMEM((N_CAT, 512), jnp.int32),
            pltpu.VMEM((2, CROWS, LANE), jnp.float32),
            pltpu.SemaphoreType.DMA,
            pltpu.SemaphoreType.DMA,
            pltpu.SemaphoreType.DMA,
        ],
    )(tab, idxt)


def _mm_body(cat_ref, w_ref, b_ref, o_ref):
    acc = b_ref[...].astype(jnp.float32)
    for i in range(N_CAT):
        acc = acc + jnp.dot(
            cat_ref[i].astype(jnp.bfloat16),
            w_ref[i].astype(jnp.bfloat16),
            preferred_element_type=jnp.float32)
    o_ref[...] = acc


def _tc_matmul(cat3, w3, b2d):
    batch = cat3.shape[1]
    bm = 1024
    return pl.pallas_call(
        _mm_body,
        grid=(batch // bm,),
        in_specs=[
            pl.BlockSpec((N_CAT, bm, LANE), lambda i: (0, i, 0)),
            pl.BlockSpec((N_CAT, LANE, OUT_DIM), lambda i: (0, 0, 0)),
            pl.BlockSpec((1, OUT_DIM), lambda i: (0, 0)),
        ],
        out_specs=pl.BlockSpec((bm, OUT_DIM), lambda i: (i, 0)),
        out_shape=jax.ShapeDtypeStruct((batch, OUT_DIM), jnp.float32),
    )(cat3, w3, b2d)


def kernel(table0, table1, table2, table3, table4, table5, table6, table7,
           W, b, prev_action):
    tables = [table0, table1, table2, table3, table4, table5, table6, table7]
    batch = prev_action.shape[0]
    tab = jnp.pad(jnp.concatenate(tables, axis=0), ((0, 0), (0, LANE - EMBED)))
    idxt = (prev_action.astype(jnp.int32)
            + jnp.arange(N_CAT, dtype=jnp.int32) * VOCAB).T
    w3 = jnp.pad(W.reshape(N_CAT, EMBED, OUT_DIM),
                 ((0, 0), (0, LANE - EMBED), (0, 0)))
    cat3 = _sc_gather(tab, idxt)
    return _tc_matmul(cat3, w3, b.reshape(1, OUT_DIM))
